# baseline (speedup 1.0000x reference)
import functools

import jax
import jax.numpy as jnp
from jax import lax
from jax.experimental import pallas as pl
from jax.experimental.pallas import tpu as pltpu, tpu_sc as plsc

VOCAB = 100000
HID = 1024
MAX_POS = 2048
N_LANG = 8
B = 4
S = 2048

NC = 2
NS = 16
NW = NC * NS
TOK = B * S
TPW = TOK // NW
CH = 32
NCHUNK = TPW // CH
NV = HID // 16


def _rsqrt16(v):
    i = lax.bitcast_convert_type(v, jnp.int32)
    i = jnp.int32(0x5F3759DF) - lax.shift_right_arithmetic(i, 1)
    y = lax.bitcast_convert_type(i, jnp.float32)
    for _ in range(3):
        y = y * (1.5 - 0.5 * v * y * y)
    return y


def _kernel_body(ids_hbm, lang_hbm, ww_hbm, wp_hbm, wl_hbm, g_hbm, bta_hbm,
                 out_hbm, idx_v, lid_v, lrows_v, g_v, bta_v, p_v,
                 w_v, sem):
    wid = lax.axis_index("s") * NC + lax.axis_index("c")
    base = wid * TPW
    bidx = base // S
    s0 = base - bidx * S

    pltpu.sync_copy(ids_hbm.at[pl.ds(base, TPW)], idx_v)
    pltpu.sync_copy(g_hbm, g_v)
    pltpu.sync_copy(bta_hbm, bta_v)
    pltpu.sync_copy(lang_hbm, lid_v)
    pltpu.async_copy(wl_hbm.at[lid_v], lrows_v, sem).wait()

    for c in range(NCHUNK):
        pltpu.async_copy(ww_hbm.at[idx_v.at[pl.ds(c * CH, CH)]], w_v,
                         sem).wait()
        pltpu.sync_copy(wp_hbm.at[pl.ds(s0 + c * CH, CH)], p_v)

        def row_body(r, _):
            def acc_body(j, carry):
                s1, s2 = carry
                col = j * 16
                x = (w_v[r, pl.ds(col, 16)] + p_v[r, pl.ds(col, 16)]
                     + lrows_v[bidx, pl.ds(col, 16)])
                w_v[r, pl.ds(col, 16)] = x
                return s1 + x, s2 + x * x

            s1, s2 = lax.fori_loop(
                0, NV, acc_body,
                (jnp.zeros((16,), jnp.float32), jnp.zeros((16,), jnp.float32)))
            mu = jnp.sum(s1) * (1.0 / HID)
            var = jnp.sum(s2) * (1.0 / HID) - mu * mu
            rstd = _rsqrt16(jnp.full((16,), var + 1e-5, jnp.float32))

            def norm_body(j, _):
                col = j * 16
                x = w_v[r, pl.ds(col, 16)]
                y = (x - mu) * rstd * g_v[pl.ds(col, 16)] + bta_v[pl.ds(col, 16)]
                w_v[r, pl.ds(col, 16)] = y
                return 0

            lax.fori_loop(0, NV, norm_body, 0)
            return 0

        lax.fori_loop(0, CH, row_body, 0)
        pltpu.sync_copy(w_v, out_hbm.at[pl.ds(base + c * CH, CH)])


@jax.jit
def _run(ids_flat, lang_pad, W_word, W_pos, W_lang, ln_gamma, ln_beta):
    mesh = plsc.VectorSubcoreMesh(core_axis_name="c", subcore_axis_name="s")
    return pl.kernel(
        _kernel_body,
        out_type=jax.ShapeDtypeStruct((TOK, HID), jnp.float32),
        mesh=mesh,
        compiler_params=pltpu.CompilerParams(needs_layout_passes=False),
        scratch_types=[
            pltpu.VMEM((TPW,), jnp.int32),
            pltpu.VMEM((N_LANG,), jnp.int32),
            pltpu.VMEM((N_LANG, HID), jnp.float32),
            pltpu.VMEM((HID,), jnp.float32),
            pltpu.VMEM((HID,), jnp.float32),
            pltpu.VMEM((CH, HID), jnp.float32),
            pltpu.VMEM((CH, HID), jnp.float32),
            pltpu.SemaphoreType.DMA,
        ],
    )(ids_flat, lang_pad, W_word, W_pos, W_lang, ln_gamma, ln_beta)


def kernel(input_ids, language_id, W_word, W_pos, W_lang, ln_gamma, ln_beta):
    ids_flat = input_ids.reshape(-1).astype(jnp.int32)
    lang_pad = jnp.concatenate(
        [language_id.astype(jnp.int32),
         jnp.zeros((N_LANG - B,), jnp.int32)])
    out = _run(ids_flat, lang_pad, W_word, W_pos, W_lang, ln_gamma, ln_beta)
    return out.reshape(B, S, HID)


# speedup vs baseline: 3.2485x; 3.2485x over previous
import jax
import jax.numpy as jnp
from jax import lax
from jax.experimental import pallas as pl
from jax.experimental.pallas import tpu as pltpu, tpu_sc as plsc

VOCAB = 100000
HID = 1024
MAX_POS = 2048
N_LANG = 8
B = 4
S = 2048

NC = 2
NS = 16
NW = NC * NS
TOK = B * S
TPW = TOK // NW
CH = 16
NCHUNK = TPW // CH
NV = HID // 16


def _rsqrt16(v):
    i = lax.bitcast_convert_type(v, jnp.int32)
    i = jnp.int32(0x5F3759DF) - lax.shift_right_arithmetic(i, 1)
    y = lax.bitcast_convert_type(i, jnp.float32)
    for _ in range(3):
        y = y * (1.5 - 0.5 * v * y * y)
    return y


def _kernel_body(ids_hbm, lang_hbm, ww_hbm, wp_hbm, wl_hbm, g_hbm, bta_hbm,
                 out_hbm, idx_v, lid_v, lrows_v, g_v, bta_v, w_v, p_v, o_v,
                 semg, semw, semp, semo):
    wid = lax.axis_index("s") * NC + lax.axis_index("c")
    base = wid * TPW
    bidx = base // S
    s0 = base - bidx * S

    pltpu.sync_copy(ids_hbm.at[pl.ds(base, TPW)], idx_v)
    pltpu.sync_copy(g_hbm, g_v)
    pltpu.sync_copy(bta_hbm, bta_v)
    pltpu.sync_copy(lang_hbm, lid_v)
    pltpu.async_copy(wl_hbm.at[lid_v], lrows_v, semg).wait()

    def issue_in(c, slot):
        pltpu.async_copy(ww_hbm.at[idx_v.at[pl.ds(c * CH, CH)]],
                         w_v.at[slot], semw.at[slot])
        pltpu.async_copy(wp_hbm.at[pl.ds(s0 + c * CH, CH)],
                         p_v.at[slot], semp.at[slot])

    issue_in(0, 0)
    issue_in(1, 1)

    def chunk_body(c, _):
        slot = lax.rem(c, 2)
        pltpu.make_async_copy(ww_hbm.at[idx_v.at[pl.ds(0, CH)]],
                              w_v.at[slot], semw.at[slot]).wait()
        pltpu.make_async_copy(wp_hbm.at[pl.ds(0, CH)],
                              p_v.at[slot], semp.at[slot]).wait()

        @pl.when(c >= 2)
        def _():
            pltpu.make_async_copy(o_v.at[slot], out_hbm.at[pl.ds(base, CH)],
                                  semo.at[slot]).wait()

        def row_body(r, _):
            zero = jnp.zeros((16,), jnp.float32)

            @plsc.parallel_loop(0, NV, unroll=8, carry=(zero, zero))
            def sums(j, carry):
                s1, s2 = carry
                col = j * 16
                x = (w_v[slot, r, pl.ds(col, 16)]
                     + p_v[slot, r, pl.ds(col, 16)]
                     + lrows_v[bidx, pl.ds(col, 16)])
                o_v[slot, r, pl.ds(col, 16)] = x
                return s1 + x, s2 + x * x

            s1, s2 = sums
            mu = jnp.sum(s1) * (1.0 / HID)
            var = jnp.sum(s2) * (1.0 / HID) - mu * mu
            rstd = _rsqrt16(jnp.full((16,), var + 1e-5, jnp.float32))

            @plsc.parallel_loop(0, NV, unroll=8)
            def norm(j):
                col = j * 16
                x = o_v[slot, r, pl.ds(col, 16)]
                o_v[slot, r, pl.ds(col, 16)] = (
                    (x - mu) * rstd * g_v[pl.ds(col, 16)]
                    + bta_v[pl.ds(col, 16)])

            return 0

        lax.fori_loop(0, CH, row_body, 0)

        pltpu.async_copy(o_v.at[slot], out_hbm.at[pl.ds(base + c * CH, CH)],
                         semo.at[slot])

        @pl.when(c + 2 < NCHUNK)
        def _():
            issue_in(c + 2, slot)

        return 0

    lax.fori_loop(0, NCHUNK, chunk_body, 0)

    pltpu.make_async_copy(o_v.at[0], out_hbm.at[pl.ds(base, CH)],
                          semo.at[0]).wait()
    pltpu.make_async_copy(o_v.at[1], out_hbm.at[pl.ds(base, CH)],
                          semo.at[1]).wait()


@jax.jit
def _run(ids_flat, lang_pad, W_word, W_pos, W_lang, ln_gamma, ln_beta):
    mesh = plsc.VectorSubcoreMesh(core_axis_name="c", subcore_axis_name="s")
    return pl.kernel(
        _kernel_body,
        out_type=jax.ShapeDtypeStruct((TOK, HID), jnp.float32),
        mesh=mesh,
        compiler_params=pltpu.CompilerParams(needs_layout_passes=False),
        scratch_types=[
            pltpu.VMEM((TPW,), jnp.int32),
            pltpu.VMEM((N_LANG,), jnp.int32),
            pltpu.VMEM((N_LANG, HID), jnp.float32),
            pltpu.VMEM((HID,), jnp.float32),
            pltpu.VMEM((HID,), jnp.float32),
            pltpu.VMEM((2, CH, HID), jnp.float32),
            pltpu.VMEM((2, CH, HID), jnp.float32),
            pltpu.VMEM((2, CH, HID), jnp.float32),
            pltpu.SemaphoreType.DMA,
            pltpu.SemaphoreType.DMA((2,)),
            pltpu.SemaphoreType.DMA((2,)),
            pltpu.SemaphoreType.DMA((2,)),
        ],
    )(ids_flat, lang_pad, W_word, W_pos, W_lang, ln_gamma, ln_beta)


def kernel(input_ids, language_id, W_word, W_pos, W_lang, ln_gamma, ln_beta):
    ids_flat = input_ids.reshape(-1).astype(jnp.int32)
    lang_pad = jnp.concatenate(
        [language_id.astype(jnp.int32),
         jnp.zeros((N_LANG - B,), jnp.int32)])
    out = _run(ids_flat, lang_pad, W_word, W_pos, W_lang, ln_gamma, ln_beta)
    return out.reshape(B, S, HID)
